# Initial kernel scaffold; baseline (speedup 1.0000x reference)
import jax
import jax.numpy as jnp
from jax.experimental import pallas as pl


def kernel(inputs, W, b, P0):
    raise NotImplementedError("write your pallas kernel here")


# speedup vs baseline: 1.8266x; 1.8266x over previous
import functools

import jax
import jax.numpy as jnp
from jax.experimental import pallas as pl

WORKER_NUM = 1000
TASK_NUM = 20000
ABILITY_NUM = 128
EDGE_TYPE = 2
COLS = TASK_NUM * EDGE_TYPE

ROW_BLOCK = 8


def _decoder_block(wf_ref, w_ref, b_ref, tau_ref, out_ref):
    x = jnp.dot(wf_ref[...], w_ref[...],
                preferred_element_type=jnp.float32) + b_ref[0, 0]
    p1 = jax.nn.sigmoid(x)
    p2 = 1.0 - p1
    lp1 = jnp.maximum(jnp.log(p1), -1e30)
    lp2 = jnp.maximum(jnp.log(p2), -1e30)
    a = lp1 - lp2
    out_ref[...] = jnp.exp(lp2 + tau_ref[...] * a)


@jax.jit
def kernel(inputs, W, b, P0):
    wf = inputs[:WORKER_NUM]
    tau = inputs[WORKER_NUM:, :EDGE_TYPE].reshape(1, COLS)
    b2 = b.reshape(1, 1)
    grid = (WORKER_NUM // ROW_BLOCK,)
    out = pl.pallas_call(
        _decoder_block,
        grid=grid,
        in_specs=[
            pl.BlockSpec((ROW_BLOCK, ABILITY_NUM), lambda i: (i, 0)),
            pl.BlockSpec((ABILITY_NUM, 1), lambda i: (0, 0)),
            pl.BlockSpec((1, 1), lambda i: (0, 0)),
            pl.BlockSpec((1, COLS), lambda i: (0, 0)),
        ],
        out_specs=pl.BlockSpec((ROW_BLOCK, COLS), lambda i: (i, 0)),
        out_shape=jax.ShapeDtypeStruct((WORKER_NUM, COLS), jnp.float32),
    )(wf, W, b2, tau)
    return out.reshape(WORKER_NUM, TASK_NUM, EDGE_TYPE)


# speedup vs baseline: 1.9231x; 1.0528x over previous
import functools

import jax
import jax.numpy as jnp
from jax.experimental import pallas as pl

WORKER_NUM = 1000
TASK_NUM = 20000
ABILITY_NUM = 128
EDGE_TYPE = 2
COLS = TASK_NUM * EDGE_TYPE

ROW_BLOCK = 40


def _decoder_block(wf_ref, w_ref, b_ref, tau_ref, out_ref):
    x = jnp.dot(wf_ref[...], w_ref[...],
                preferred_element_type=jnp.float32) + b_ref[0, 0]
    p1 = jax.nn.sigmoid(x)
    p2 = 1.0 - p1
    lp1 = jnp.maximum(jnp.log(p1), -1e30)
    lp2 = jnp.maximum(jnp.log(p2), -1e30)
    a = lp1 - lp2
    out_ref[...] = jnp.exp(lp2 + tau_ref[...] * a)


@jax.jit
def kernel(inputs, W, b, P0):
    wf = inputs[:WORKER_NUM]
    tau = inputs[WORKER_NUM:, :EDGE_TYPE].reshape(1, COLS)
    b2 = b.reshape(1, 1)
    grid = (WORKER_NUM // ROW_BLOCK,)
    out = pl.pallas_call(
        _decoder_block,
        grid=grid,
        in_specs=[
            pl.BlockSpec((ROW_BLOCK, ABILITY_NUM), lambda i: (i, 0)),
            pl.BlockSpec((ABILITY_NUM, 1), lambda i: (0, 0)),
            pl.BlockSpec((1, 1), lambda i: (0, 0)),
            pl.BlockSpec((1, COLS), lambda i: (0, 0)),
        ],
        out_specs=pl.BlockSpec((ROW_BLOCK, COLS), lambda i: (i, 0)),
        out_shape=jax.ShapeDtypeStruct((WORKER_NUM, COLS), jnp.float32),
    )(wf, W, b2, tau)
    return out.reshape(WORKER_NUM, TASK_NUM, EDGE_TYPE)
